# baseline (speedup 1.0000x reference)
import functools

import jax
import jax.numpy as jnp
from jax import lax
from jax.experimental import pallas as pl
from jax.experimental.pallas import tpu as pltpu
from jax.experimental.pallas import tpu_sc as plsc

_B, _H, _N = 32, 32, 32768
_ROWS = _B * _H
_NC, _NS, _L = 2, 16, 16
_NW = _NC * _NS
_RPW = _ROWS // _NW
_NV = _N // _L
_NBLK = _NV // 16
_K = 10
_NEG = float("-inf")


def _merge_top16(t, v):
    sa, _ = plsc.sort_key_val(t, t, descending=False)
    sb, _ = plsc.sort_key_val(v, v, descending=True)
    return jnp.maximum(sa, sb)


def _lane(v, k, iota, fill):
    return jnp.max(jnp.where(iota == k, v, fill))


def _sc_body(x_hbm, o_hbm, bufs, cmax_buf, hot_buf, kept_idxs, seg_stages,
             zeros_sh, sems_in, sems_patch, sem_zero):
    wid = lax.axis_index("s") * _NC + lax.axis_index("c")
    sid = lax.axis_index("s")
    iota = lax.iota(jnp.int32, _L)
    zerov = jnp.zeros((_L,), jnp.float32)
    zeroi = jnp.zeros((_L,), jnp.int32)
    neginf = jnp.full((_L,), _NEG, jnp.float32)

    row0 = wid * _RPW
    row_last = row0 + _RPW - 1

    for p in range(2):
        kept_idxs[p][...] = zeroi
        for k in range(_K):
            for u in range(8):
                seg_stages[p][k, pl.ds(u * _L, _L)] = zerov

    @pl.when(sid == 0)
    def _():
        def ztpl_body(i, c):
            pltpu.sync_copy(seg_stages[0].at[0], zeros_sh.at[i])
            return c
        lax.fori_loop(0, _N // 128, ztpl_body, 0)

    plsc.subcore_barrier()

    for p in range(2):
        for k in range(_K):
            pltpu.async_copy(
                seg_stages[p].at[k],
                o_hbm.at[row0].at[0], sems_patch[p])
    pltpu.async_copy(x_hbm.at[row0], bufs[0], sems_in[0])

    def process(row, cur, nxt, sem_cur, sem_nxt, kept_idx, seg_stage,
                sem_patch):
        pltpu.async_copy(zeros_sh, o_hbm.at[row], sem_zero)
        pltpu.make_async_copy(x_hbm.at[row], cur, sem_cur).wait()
        nrow = jnp.minimum(row + 1, row_last)
        pltpu.async_copy(x_hbm.at[nrow], nxt, sem_nxt)

        def blk_body(i, c):
            base = i * 256
            vs = [cur[pl.ds(base + u * _L, _L)] for u in range(16)]
            while len(vs) > 1:
                vs = [jnp.maximum(vs[2 * j], vs[2 * j + 1])
                      for j in range(len(vs) // 2)]
            cmax_buf[pl.ds(i * _L, _L)] = vs[0]
            return c

        lax.fori_loop(0, _NBLK, blk_body, 0)

        def fold_body(k, accs):
            out = []
            for j in range(8):
                cm = cmax_buf[pl.ds((k * 8 + j) * _L, _L)]
                out.append(jnp.maximum(accs[j], cm))
            return tuple(out)

        maccs = lax.fori_loop(0, 16, fold_body, (neginf,) * 8)

        top = maccs[0]
        for j in range(1, 8):
            top = _merge_top16(top, maccs[j])
        tops, _ = plsc.sort_key_val(top, top, descending=True)
        t00 = _lane(tops, _K - 1, iota, neginf)
        t00v = jnp.full((_L,), t00, jnp.float32)

        def hot_body(i, ptr):
            cm = cmax_buf[pl.ds(i * _L, _L)]
            msk = cm >= t00v
            mi = msk.astype(jnp.int32)
            pos = ptr + plsc.cumsum(mi) - 1
            base_vec = i * 256 + iota
            plsc.store_scatter(hot_buf, [pos], base_vec, mask=msk)
            return ptr + jnp.sum(mi)

        nh = lax.fori_loop(0, _NBLK, hot_body, 0)
        ngrp = (nh + _L - 1) // _L
        nhv = jnp.full((_L,), nh, jnp.int32)

        def top_body(g, top):
            hv = hot_buf[pl.ds(g * _L, _L)]
            valid = (iota + g * _L) < nhv
            hv = jnp.where(valid, hv, 0)
            for u in range(16):
                idxv = hv + 16 * u
                v = plsc.load_gather(cur, [idxv])
                v = jnp.where(valid, v, neginf)
                top = _merge_top16(top, v)
            return top

        top = lax.fori_loop(0, ngrp, top_body, neginf)
        tsort, _ = plsc.sort_key_val(top, top, descending=True)
        t = _lane(tsort, _K - 1, iota, neginf)
        tv = jnp.full((_L,), t, jnp.float32)

        for _k in range(_K):
            pltpu.make_async_copy(
                seg_stage.at[_k],
                o_hbm.at[row].at[0], sem_patch).wait()

        def gt_body(g, kptr):
            hv = hot_buf[pl.ds(g * _L, _L)]
            valid = (iota + g * _L) < nhv
            hv = jnp.where(valid, hv, 0)
            for u in range(16):
                idxv = hv + 16 * u
                v = plsc.load_gather(cur, [idxv])
                m = jnp.logical_and(v > tv, valid)
                mi = m.astype(jnp.int32)
                pos = kptr + plsc.cumsum(mi) - 1
                plsc.store_scatter(kept_idx, [pos], idxv, mask=m)
                kptr = kptr + jnp.sum(mi)
            return kptr

        cnt_gt = lax.fori_loop(0, ngrp, gt_body, 0)

        def tie_body(k, carry):
            last, kptr = carry
            lastv = jnp.full((_L,), last, jnp.int32)

            def find_body(g, best):
                hv = hot_buf[pl.ds(g * _L, _L)]
                valid = (iota + g * _L) < nhv
                hv = jnp.where(valid, hv, 0)
                for u in range(16):
                    idxv = hv + 16 * u
                    v = plsc.load_gather(cur, [idxv])
                    eq = jnp.logical_and(
                        jnp.logical_and(v == tv, valid), idxv > lastv)
                    cand = jnp.where(eq, idxv, _N)
                    best = jnp.minimum(best, jnp.min(cand))
                return best

            best = lax.fori_loop(0, ngrp, find_body, _N)
            bv = jnp.full((_L,), best, jnp.int32)
            m0 = jnp.logical_and(iota == 0, bv < _N)
            plsc.store_scatter(
                kept_idx, [jnp.full((_L,), kptr, jnp.int32)], bv, mask=m0)
            return (best, kptr + 1)

        last_tie, _unused = lax.fori_loop(
            0, _K - cnt_gt, tie_body, (-1, 0 + cnt_gt))
        lastv = jnp.full((_L,), last_tie, jnp.int32)

        kraw = kept_idx[...]
        bases = []
        for k in range(_K):
            bk = _lane(kraw, k, iota, -1)
            seg = bk >> 7
            base = seg << 7
            bases.append(seg)
            for u in range(8):
                vseg = cur[pl.ds(base + u * _L, _L)]
                idxs = base + u * _L + iota
                keep = jnp.logical_or(
                    vseg > tv,
                    jnp.logical_and(vseg == tv, idxs <= lastv))
                seg_stage[k, pl.ds(u * _L, _L)] = jnp.where(
                    keep, vseg, zerov)

        pltpu.make_async_copy(zeros_sh, o_hbm.at[row], sem_zero).wait()
        for k in range(_K):
            pltpu.async_copy(
                seg_stage.at[k],
                o_hbm.at[row].at[bases[k]], sem_patch)

    def pair_body(k, c):
        r = row0 + 2 * k
        process(r, bufs[0], bufs[1], sems_in[0], sems_in[1],
                kept_idxs[0], seg_stages[0], sems_patch[0])
        process(r + 1, bufs[1], bufs[0], sems_in[1], sems_in[0],
                kept_idxs[1], seg_stages[1], sems_patch[1])
        return c

    lax.fori_loop(0, _RPW // 2, pair_body, 0)

    pltpu.make_async_copy(x_hbm.at[row_last], bufs[0], sems_in[0]).wait()
    for p in range(2):
        for k in range(_K):
            pltpu.make_async_copy(
                seg_stages[p].at[k],
                o_hbm.at[row_last].at[0], sems_patch[p]).wait()


_sc_topk = functools.partial(
    pl.kernel,
    out_type=jax.ShapeDtypeStruct((_ROWS, _N // 128, 128), jnp.float32),
    mesh=plsc.VectorSubcoreMesh(
        core_axis_name="c", subcore_axis_name="s",
        num_cores=_NC, num_subcores=_NS),
    scratch_types=[
        (pltpu.VMEM((_N,), jnp.float32),) * 2,
        pltpu.VMEM((_NV,), jnp.float32),
        pltpu.VMEM((_NV,), jnp.int32),
        (pltpu.VMEM((_L,), jnp.int32),) * 2,
        (pltpu.VMEM((_K, 128), jnp.float32),) * 2,
        pltpu.VMEM_SHARED((_N // 128, 128), jnp.float32),
        (pltpu.SemaphoreType.DMA,) * 2,
        (pltpu.SemaphoreType.DMA,) * 2,
        pltpu.SemaphoreType.DMA,
    ],
    compiler_params=pltpu.CompilerParams(needs_layout_passes=False),
)(_sc_body)


@jax.jit
def kernel(inputs):
    x2 = inputs.reshape(_ROWS, _N)
    out = _sc_topk(x2)
    return out.reshape(inputs.shape)


# speedup vs baseline: 2.6217x; 2.6217x over previous
import functools

import jax
import jax.numpy as jnp
from jax import lax
from jax.experimental import pallas as pl
from jax.experimental.pallas import tpu as pltpu
from jax.experimental.pallas import tpu_sc as plsc

_B, _H, _N = 32, 32, 32768
_ROWS = _B * _H
_NC, _NS, _L = 2, 16, 16
_NW = _NC * _NS
_RPW = _ROWS // _NW
_NV = _N // _L
_NBLK = _NV // 16
_K = 10
_NEG = float("-inf")


def _merge_top16(t, v):
    sa, _ = plsc.sort_key_val(t, t, descending=False)
    sb, _ = plsc.sort_key_val(v, v, descending=True)
    return jnp.maximum(sa, sb)


def _lane(v, k, iota, fill):
    return jnp.max(jnp.where(iota == k, v, fill))


def _sc_body(x_hbm, o_hbm, bufs, out_buf, cmax_buf, hot_buf, sg_buf,
             kepts, sems_in, sem_out):
    wid = lax.axis_index("s") * _NC + lax.axis_index("c")
    iota = lax.iota(jnp.int32, _L)
    zerov = jnp.zeros((_L,), jnp.float32)
    zeroi = jnp.zeros((_L,), jnp.int32)
    neginf = jnp.full((_L,), _NEG, jnp.float32)

    row0 = wid * _RPW
    row_last = row0 + _RPW - 1

    def zero_body(i, c):
        for u in range(8):
            out_buf[pl.ds((i * 8 + u) * _L, _L)] = zerov
        return c

    lax.fori_loop(0, _NV // 8, zero_body, 0)
    kepts[0][...] = zeroi
    kepts[1][...] = zeroi
    pltpu.async_copy(x_hbm.at[row0], bufs[0], sems_in[0])
    pltpu.async_copy(out_buf, o_hbm.at[row0], sem_out)

    def process(row, cur, nxt, sem_cur, sem_nxt, kept_cur, kept_prev):
        pltpu.make_async_copy(x_hbm.at[row], cur, sem_cur).wait()
        nrow = jnp.minimum(row + 1, row_last)
        pltpu.async_copy(x_hbm.at[nrow], nxt, sem_nxt)

        def blk_body(k, accs):
            out = []
            for j in range(8):
                i = k * 8 + j
                base = i * 256
                vs = [cur[pl.ds(base + u * _L, _L)] for u in range(16)]
                while len(vs) > 1:
                    vs = [jnp.maximum(vs[2 * m], vs[2 * m + 1])
                          for m in range(len(vs) // 2)]
                cmax_buf[pl.ds(i * _L, _L)] = vs[0]
                out.append(jnp.maximum(accs[j], vs[0]))
            return tuple(out)

        maccs = lax.fori_loop(0, 16, blk_body, (neginf,) * 8)

        tops = [_merge_top16(maccs[2 * j], maccs[2 * j + 1])
                for j in range(4)]
        tops = [_merge_top16(tops[0], tops[1]),
                _merge_top16(tops[2], tops[3])]
        top = _merge_top16(tops[0], tops[1])
        tops, _ = plsc.sort_key_val(top, top, descending=True)
        t00 = _lane(tops, _K - 1, iota, neginf)
        t00v = jnp.full((_L,), t00, jnp.float32)

        sgptr = 0
        for j in range(8):
            m = maccs[j] >= t00v
            mi = m.astype(jnp.int32)
            pos = sgptr + plsc.cumsum(mi) - 1
            plsc.store_scatter(sg_buf, [pos], 16 * j + iota, mask=m)
            sgptr = sgptr + jnp.sum(mi)
        nsg = sgptr
        nsgv = jnp.full((_L,), nsg, jnp.int32)

        def sgg_body(gg, hotptr):
            sgv = sg_buf[pl.ds(gg * _L, _L)]
            validsg = (iota + gg * _L) < nsgv
            sgv = jnp.where(validsg, sgv, 0)
            for k in range(16):
                cidx = 128 * k + sgv
                cm = plsc.load_gather(cmax_buf, [cidx])
                m = jnp.logical_and(cm >= t00v, validsg)
                basev = ((cidx >> 4) << 8) + (cidx & 15)
                mi = m.astype(jnp.int32)
                pos = hotptr + plsc.cumsum(mi) - 1
                plsc.store_scatter(hot_buf, [pos], basev, mask=m)
                hotptr = hotptr + jnp.sum(mi)
            return hotptr

        nsgg = (nsg + _L - 1) // _L
        nh = lax.fori_loop(0, nsgg, sgg_body, 0)
        ngrp = (nh + _L - 1) // _L
        nhv = jnp.full((_L,), nh, jnp.int32)

        def top_body(g, top):
            hv = hot_buf[pl.ds(g * _L, _L)]
            valid = (iota + g * _L) < nhv
            hv = jnp.where(valid, hv, 0)
            vs = []
            for u in range(16):
                idxv = hv + 16 * u
                v = plsc.load_gather(cur, [idxv])
                vs.append(jnp.where(valid, v, neginf))
            while len(vs) > 1:
                vs = [_merge_top16(vs[2 * m], vs[2 * m + 1])
                      for m in range(len(vs) // 2)]
            return _merge_top16(top, vs[0])

        top = lax.fori_loop(0, ngrp, top_body, neginf)
        tsort, _ = plsc.sort_key_val(top, top, descending=True)
        t = _lane(tsort, _K - 1, iota, neginf)
        tv = jnp.full((_L,), t, jnp.float32)

        pltpu.make_async_copy(out_buf, o_hbm.at[row], sem_out).wait()
        kprev = kept_prev[...]
        plsc.store_scatter(out_buf, [kprev], zerov, mask=iota < _K)

        def gt_body(g, kptr):
            hv = hot_buf[pl.ds(g * _L, _L)]
            valid = (iota + g * _L) < nhv
            hv = jnp.where(valid, hv, 0)
            for u in range(16):
                idxv = hv + 16 * u
                v = plsc.load_gather(cur, [idxv])
                m = jnp.logical_and(v > tv, valid)
                mi = m.astype(jnp.int32)
                pos = kptr + plsc.cumsum(mi) - 1
                plsc.store_scatter(kept_cur, [pos], idxv, mask=m)
                plsc.store_scatter(out_buf, [idxv], v, mask=m)
                kptr = kptr + jnp.sum(mi)
            return kptr

        cnt_gt = lax.fori_loop(0, ngrp, gt_body, 0)

        def tie_body(k, carry):
            last, kptr = carry
            lastv = jnp.full((_L,), last, jnp.int32)

            def find_body(g, best):
                hv = hot_buf[pl.ds(g * _L, _L)]
                valid = (iota + g * _L) < nhv
                hv = jnp.where(valid, hv, 0)
                for u in range(16):
                    idxv = hv + 16 * u
                    v = plsc.load_gather(cur, [idxv])
                    eq = jnp.logical_and(
                        jnp.logical_and(v == tv, valid), idxv > lastv)
                    cand = jnp.where(eq, idxv, _N)
                    best = jnp.minimum(best, jnp.min(cand))
                return best

            best = lax.fori_loop(0, ngrp, find_body, _N)
            bv = jnp.full((_L,), best, jnp.int32)
            m0 = jnp.logical_and(iota == 0, bv < _N)
            plsc.store_scatter(out_buf, [bv], tv, mask=m0)
            plsc.store_scatter(
                kept_cur, [jnp.full((_L,), kptr, jnp.int32)], bv, mask=m0)
            return (best, kptr + 1)

        lax.fori_loop(0, _K - cnt_gt, tie_body, (-1, 0 + cnt_gt))

        pltpu.async_copy(out_buf, o_hbm.at[row], sem_out)

    def pair_body(k, c):
        r = row0 + 2 * k
        process(r, bufs[0], bufs[1], sems_in[0], sems_in[1],
                kepts[0], kepts[1])
        process(r + 1, bufs[1], bufs[0], sems_in[1], sems_in[0],
                kepts[1], kepts[0])
        return c

    lax.fori_loop(0, _RPW // 2, pair_body, 0)

    pltpu.make_async_copy(x_hbm.at[row_last], bufs[0], sems_in[0]).wait()
    pltpu.make_async_copy(out_buf, o_hbm.at[row_last], sem_out).wait()


_sc_topk = functools.partial(
    pl.kernel,
    out_type=jax.ShapeDtypeStruct((_ROWS, _N), jnp.float32),
    mesh=plsc.VectorSubcoreMesh(
        core_axis_name="c", subcore_axis_name="s",
        num_cores=_NC, num_subcores=_NS),
    scratch_types=[
        (pltpu.VMEM((_N,), jnp.float32),) * 2,
        pltpu.VMEM((_N,), jnp.float32),
        pltpu.VMEM((_NV,), jnp.float32),
        pltpu.VMEM((_NV,), jnp.int32),
        pltpu.VMEM((128,), jnp.int32),
        (pltpu.VMEM((_L,), jnp.int32),) * 2,
        (pltpu.SemaphoreType.DMA,) * 2,
        pltpu.SemaphoreType.DMA,
    ],
    compiler_params=pltpu.CompilerParams(needs_layout_passes=False),
)(_sc_body)


@jax.jit
def kernel(inputs):
    x2 = inputs.reshape(_ROWS, _N)
    out = _sc_topk(x2)
    return out.reshape(inputs.shape)


# speedup vs baseline: 2.7470x; 1.0478x over previous
import functools

import jax
import jax.numpy as jnp
from jax import lax
from jax.experimental import pallas as pl
from jax.experimental.pallas import tpu as pltpu
from jax.experimental.pallas import tpu_sc as plsc

_B, _H, _N = 32, 32, 32768
_ROWS = _B * _H
_NC, _NS, _L = 2, 16, 16
_NW = _NC * _NS
_RPW = _ROWS // _NW
_NV = _N // _L
_NBLK = _NV // 16
_K = 10
_NEG = float("-inf")


def _merge_top16(t, v):
    sa, _ = plsc.sort_key_val(t, t, descending=False)
    sb, _ = plsc.sort_key_val(v, v, descending=True)
    return jnp.maximum(sa, sb)


def _lane(v, k, iota, fill):
    return jnp.max(jnp.where(iota == k, v, fill))


def _sc_body(x_hbm, o_hbm, bufs, out_buf, cmax_buf, hot_buf, sg_buf,
             kepts, sems_in, sem_out):
    wid = lax.axis_index("s") * _NC + lax.axis_index("c")
    iota = lax.iota(jnp.int32, _L)
    zerov = jnp.zeros((_L,), jnp.float32)
    zeroi = jnp.zeros((_L,), jnp.int32)
    neginf = jnp.full((_L,), _NEG, jnp.float32)

    row0 = wid * _RPW
    row_last = row0 + _RPW - 1

    def zero_body(i, c):
        for u in range(8):
            out_buf[pl.ds((i * 8 + u) * _L, _L)] = zerov
        return c

    lax.fori_loop(0, _NV // 8, zero_body, 0)
    kepts[0][...] = zeroi
    kepts[1][...] = zeroi
    pltpu.async_copy(x_hbm.at[row0], bufs[0], sems_in[0])
    pltpu.async_copy(out_buf, o_hbm.at[row0], sem_out)

    def process(row, cur, nxt, sem_cur, sem_nxt, kept_cur, kept_prev):
        pltpu.make_async_copy(x_hbm.at[row], cur, sem_cur).wait()
        nrow = jnp.minimum(row + 1, row_last)
        pltpu.async_copy(x_hbm.at[nrow], nxt, sem_nxt)

        def blk_body(k, accs):
            out = []
            for j in range(8):
                i = k * 8 + j
                base = i * 256
                vs = [cur[pl.ds(base + u * _L, _L)] for u in range(16)]
                while len(vs) > 1:
                    vs = [jnp.maximum(vs[2 * m], vs[2 * m + 1])
                          for m in range(len(vs) // 2)]
                cmax_buf[pl.ds(i * _L, _L)] = vs[0]
                out.append(jnp.maximum(accs[j], vs[0]))
            return tuple(out)

        maccs = lax.fori_loop(0, 16, blk_body, (neginf,) * 8)

        tops = [_merge_top16(maccs[2 * j], maccs[2 * j + 1])
                for j in range(4)]
        tops = [_merge_top16(tops[0], tops[1]),
                _merge_top16(tops[2], tops[3])]
        top = _merge_top16(tops[0], tops[1])
        tops, _ = plsc.sort_key_val(top, top, descending=True)
        t00 = _lane(tops, _K - 1, iota, neginf)
        t00v = jnp.full((_L,), t00, jnp.float32)

        sgptr = 0
        for j in range(8):
            m = maccs[j] >= t00v
            mi = m.astype(jnp.int32)
            pos = sgptr + plsc.cumsum(mi) - 1
            plsc.store_scatter(sg_buf, [pos], 16 * j + iota, mask=m)
            sgptr = sgptr + jnp.sum(mi)
        nsg = sgptr
        nsgv = jnp.full((_L,), nsg, jnp.int32)

        def sgg_body(gg, hotptr):
            sgv = sg_buf[pl.ds(gg * _L, _L)]
            validsg = (iota + gg * _L) < nsgv
            sgv = jnp.where(validsg, sgv, 0)
            for k in range(16):
                cidx = 128 * k + sgv
                cm = plsc.load_gather(cmax_buf, [cidx])
                m = jnp.logical_and(cm >= t00v, validsg)
                basev = ((cidx >> 4) << 8) + (cidx & 15)
                mi = m.astype(jnp.int32)
                pos = hotptr + plsc.cumsum(mi) - 1
                plsc.store_scatter(hot_buf, [pos], basev, mask=m)
                hotptr = hotptr + jnp.sum(mi)
            return hotptr

        nsgg = (nsg + _L - 1) // _L
        nh = lax.fori_loop(0, nsgg, sgg_body, 0)
        ngrp = (nh + _L - 1) // _L
        nhv = jnp.full((_L,), nh, jnp.int32)

        def top_body(g, top):
            hv = hot_buf[pl.ds(g * _L, _L)]
            valid = (iota + g * _L) < nhv
            hv = jnp.where(valid, hv, 0)
            vs = []
            for u in range(16):
                idxv = hv + 16 * u
                v = plsc.load_gather(cur, [idxv])
                vs.append(jnp.where(valid, v, neginf))
            while len(vs) > 1:
                vs = [_merge_top16(vs[2 * m], vs[2 * m + 1])
                      for m in range(len(vs) // 2)]
            return _merge_top16(top, vs[0])

        top = lax.fori_loop(0, ngrp, top_body, neginf)
        tsort, _ = plsc.sort_key_val(top, top, descending=True)
        t = _lane(tsort, _K - 1, iota, neginf)
        tv = jnp.full((_L,), t, jnp.float32)

        pltpu.make_async_copy(out_buf, o_hbm.at[row], sem_out).wait()
        kprev = kept_prev[...]
        plsc.store_scatter(out_buf, [kprev], zerov, mask=iota < _K)

        def gt_body(g, carry):
            kptr, mineq = carry
            hv = hot_buf[pl.ds(g * _L, _L)]
            valid = (iota + g * _L) < nhv
            hv = jnp.where(valid, hv, 0)
            for u in range(16):
                idxv = hv + 16 * u
                v = plsc.load_gather(cur, [idxv])
                m = jnp.logical_and(v > tv, valid)
                mi = m.astype(jnp.int32)
                pos = kptr + plsc.cumsum(mi) - 1
                plsc.store_scatter(kept_cur, [pos], idxv, mask=m)
                plsc.store_scatter(out_buf, [idxv], v, mask=m)
                kptr = kptr + jnp.sum(mi)
                eq = jnp.logical_and(v == tv, valid)
                cand = jnp.where(eq, idxv, _N)
                mineq = jnp.minimum(mineq, jnp.min(cand))
            return (kptr, mineq)

        cnt_gt, mineq = lax.fori_loop(0, ngrp, gt_body, (0, _N))

        bv0 = jnp.full((_L,), mineq, jnp.int32)
        m00 = jnp.logical_and(iota == 0, bv0 < _N)
        plsc.store_scatter(out_buf, [bv0], tv, mask=m00)
        plsc.store_scatter(
            kept_cur, [jnp.full((_L,), cnt_gt, jnp.int32)], bv0, mask=m00)

        def tie_body(k, carry):
            last, kptr = carry
            lastv = jnp.full((_L,), last, jnp.int32)

            def find_body(g, best):
                hv = hot_buf[pl.ds(g * _L, _L)]
                valid = (iota + g * _L) < nhv
                hv = jnp.where(valid, hv, 0)
                for u in range(16):
                    idxv = hv + 16 * u
                    v = plsc.load_gather(cur, [idxv])
                    eq = jnp.logical_and(
                        jnp.logical_and(v == tv, valid), idxv > lastv)
                    cand = jnp.where(eq, idxv, _N)
                    best = jnp.minimum(best, jnp.min(cand))
                return best

            best = lax.fori_loop(0, ngrp, find_body, _N)
            bv = jnp.full((_L,), best, jnp.int32)
            m0 = jnp.logical_and(iota == 0, bv < _N)
            plsc.store_scatter(out_buf, [bv], tv, mask=m0)
            plsc.store_scatter(
                kept_cur, [jnp.full((_L,), kptr, jnp.int32)], bv, mask=m0)
            return (best, kptr + 1)

        lax.fori_loop(0, _K - cnt_gt - 1, tie_body, (mineq, cnt_gt + 1))

        pltpu.async_copy(out_buf, o_hbm.at[row], sem_out)

    def pair_body(k, c):
        r = row0 + 2 * k
        process(r, bufs[0], bufs[1], sems_in[0], sems_in[1],
                kepts[0], kepts[1])
        process(r + 1, bufs[1], bufs[0], sems_in[1], sems_in[0],
                kepts[1], kepts[0])
        return c

    lax.fori_loop(0, _RPW // 2, pair_body, 0)

    pltpu.make_async_copy(x_hbm.at[row_last], bufs[0], sems_in[0]).wait()
    pltpu.make_async_copy(out_buf, o_hbm.at[row_last], sem_out).wait()


_sc_topk = functools.partial(
    pl.kernel,
    out_type=jax.ShapeDtypeStruct((_ROWS, _N), jnp.float32),
    mesh=plsc.VectorSubcoreMesh(
        core_axis_name="c", subcore_axis_name="s",
        num_cores=_NC, num_subcores=_NS),
    scratch_types=[
        (pltpu.VMEM((_N,), jnp.float32),) * 2,
        pltpu.VMEM((_N,), jnp.float32),
        pltpu.VMEM((_NV,), jnp.float32),
        pltpu.VMEM((_NV,), jnp.int32),
        pltpu.VMEM((128,), jnp.int32),
        (pltpu.VMEM((_L,), jnp.int32),) * 2,
        (pltpu.SemaphoreType.DMA,) * 2,
        pltpu.SemaphoreType.DMA,
    ],
    compiler_params=pltpu.CompilerParams(needs_layout_passes=False),
)(_sc_body)


@jax.jit
def kernel(inputs):
    x2 = inputs.reshape(_ROWS, _N)
    out = _sc_topk(x2)
    return out.reshape(inputs.shape)
